# Initial kernel scaffold; baseline (speedup 1.0000x reference)
import functools

import numpy as np
import jax
import jax.numpy as jnp
from jax import lax
from jax.experimental import pallas as pl
from jax.experimental.pallas import tpu as pltpu
from jax.experimental.pallas import tpu_sc as plsc

_VOCAB = 100000
_NUM_DEPEND = 1000
_DIM = 64
_NUM_ROWS = 1 + _VOCAB + (_NUM_DEPEND + 1)
_RATE = 0.1
_KEEP = 1.0 - _RATE
_BATCH = 16384
_HIST = 50
_TOTAL = _BATCH * _HIST

_NC = 2
_NS = 16
_NW = _NC * _NS
_BPW = _TOTAL // _NW
_CHUNK = 128
_NCHUNK = _BPW // _CHUNK


def _np_threefry2x32(k0, k1, x0, x1):
    def rotl(x, d):
        return ((x << np.uint32(d)) | (x >> np.uint32(32 - d))).astype(np.uint32)
    rot = [(13, 15, 26, 6), (17, 29, 16, 24)]
    ks = [np.uint32(k0), np.uint32(k1),
          np.uint32(k0) ^ np.uint32(k1) ^ np.uint32(0x1BD11BDA)]
    x = [(x0 + ks[0]).astype(np.uint32), (x1 + ks[1]).astype(np.uint32)]
    order = [(rot[0], 1, 2, 1), (rot[1], 2, 0, 2), (rot[0], 0, 1, 3),
             (rot[1], 1, 2, 4), (rot[0], 2, 0, 5)]
    for rots, a, b, c in order:
        for r in rots:
            x[0] = (x[0] + x[1]).astype(np.uint32)
            x[1] = x[0] ^ rotl(x[1], r)
        x[0] = (x[0] + ks[a]).astype(np.uint32)
        x[1] = (x[1] + ks[b] + np.uint32(c)).astype(np.uint32)
    return x[0], x[1]


def _dropout_scale() -> np.ndarray:
    n = _TOTAL * _DIM
    i = np.arange(n, dtype=np.uint64)
    hi = (i >> np.uint64(32)).astype(np.uint32)
    lo = (i & np.uint64(0xFFFFFFFF)).astype(np.uint32)
    b1, b2 = _np_threefry2x32(0, 42, hi, lo)
    bits = b1 ^ b2
    fb = (bits >> np.uint32(9)) | np.uint32(0x3F800000)
    u = fb.view(np.float32) - np.float32(1.0)
    keep = u < np.float32(_KEEP)
    return np.where(keep, np.float32(1.0 / _KEEP), np.float32(0.0)).reshape(
        _TOTAL, _DIM)


_SCALE = _dropout_scale()

_mesh = plsc.VectorSubcoreMesh(core_axis_name="c", subcore_axis_name="s")


@functools.partial(
    pl.kernel,
    out_type=jax.ShapeDtypeStruct((_TOTAL, _DIM), jnp.float32),
    mesh=_mesh,
    scratch_types=[
        pltpu.VMEM((_NCHUNK, _CHUNK), jnp.int32),
        pltpu.VMEM((_CHUNK, _DIM), jnp.float32),
        pltpu.VMEM((_CHUNK, _DIM), jnp.float32),
        pltpu.SemaphoreType.DMA,
    ],
)
def _gather_dropout(w_hbm, ids_hbm, scale_hbm, out_hbm,
                    idx_v, rows_v, scale_v, sem):
    wid = lax.axis_index("s") * _NC + lax.axis_index("c")
    base = wid * _BPW
    pltpu.sync_copy(ids_hbm.at[wid], idx_v)

    def chunk(g, carry):
        off = base + g * _CHUNK
        pltpu.async_copy(w_hbm.at[idx_v.at[g]], rows_v, sem).wait()
        pltpu.sync_copy(scale_hbm.at[pl.ds(off, _CHUNK)], scale_v)

        def row(r, c2):
            for k in range(_DIM // 16):
                sl = pl.ds(k * 16, 16)
                rows_v[r, sl] = rows_v[r, sl] * scale_v[r, sl]
            return c2

        lax.fori_loop(0, _CHUNK, row, 0)
        pltpu.sync_copy(rows_v, out_hbm.at[pl.ds(off, _CHUNK)])
        return carry

    lax.fori_loop(0, _NCHUNK, chunk, 0)


def kernel(inputs, w):
    ids = jnp.reshape(inputs, (_NW, _NCHUNK, _CHUNK)).astype(jnp.int32)
    scale = jnp.asarray(_SCALE)
    out = _gather_dropout(w, ids, scale)
    return jnp.reshape(out, (_BATCH, _HIST, _DIM))


# speedup vs baseline: 2.7707x; 2.7707x over previous
import functools

import numpy as np
import jax
import jax.numpy as jnp
from jax import lax
from jax.experimental import pallas as pl
from jax.experimental.pallas import tpu as pltpu
from jax.experimental.pallas import tpu_sc as plsc

_VOCAB = 100000
_NUM_DEPEND = 1000
_DIM = 64
_NUM_ROWS = 1 + _VOCAB + (_NUM_DEPEND + 1)
_RATE = 0.1
_KEEP = 1.0 - _RATE
_BATCH = 16384
_HIST = 50
_TOTAL = _BATCH * _HIST

_NC = 2
_NS = 16
_NW = _NC * _NS
_BPW = _TOTAL // _NW
_CHUNK = 128
_NCHUNK = _BPW // _CHUNK


def _np_threefry2x32(k0, k1, x0, x1):
    def rotl(x, d):
        return ((x << np.uint32(d)) | (x >> np.uint32(32 - d))).astype(np.uint32)
    rot = [(13, 15, 26, 6), (17, 29, 16, 24)]
    ks = [np.uint32(k0), np.uint32(k1),
          np.uint32(k0) ^ np.uint32(k1) ^ np.uint32(0x1BD11BDA)]
    x = [(x0 + ks[0]).astype(np.uint32), (x1 + ks[1]).astype(np.uint32)]
    order = [(rot[0], 1, 2, 1), (rot[1], 2, 0, 2), (rot[0], 0, 1, 3),
             (rot[1], 1, 2, 4), (rot[0], 2, 0, 5)]
    for rots, a, b, c in order:
        for r in rots:
            x[0] = (x[0] + x[1]).astype(np.uint32)
            x[1] = x[0] ^ rotl(x[1], r)
        x[0] = (x[0] + ks[a]).astype(np.uint32)
        x[1] = (x[1] + ks[b] + np.uint32(c)).astype(np.uint32)
    return x[0], x[1]


def _dropout_scale() -> np.ndarray:
    n = _TOTAL * _DIM
    i = np.arange(n, dtype=np.uint64)
    hi = (i >> np.uint64(32)).astype(np.uint32)
    lo = (i & np.uint64(0xFFFFFFFF)).astype(np.uint32)
    b1, b2 = _np_threefry2x32(0, 42, hi, lo)
    bits = b1 ^ b2
    fb = (bits >> np.uint32(9)) | np.uint32(0x3F800000)
    u = fb.view(np.float32) - np.float32(1.0)
    keep = u < np.float32(_KEEP)
    return np.where(keep, np.float32(1.0 / _KEEP), np.float32(0.0)).reshape(
        _TOTAL, _DIM)


_SCALE = _dropout_scale()

_mesh = plsc.VectorSubcoreMesh(core_axis_name="c", subcore_axis_name="s")


@functools.partial(
    pl.kernel,
    out_type=jax.ShapeDtypeStruct((_TOTAL, _DIM), jnp.float32),
    mesh=_mesh,
    compiler_params=pltpu.CompilerParams(use_tc_tiling_on_sc=False),
    scratch_types=[
        pltpu.VMEM((_NCHUNK, _CHUNK), jnp.int32),
        pltpu.VMEM((_CHUNK, _DIM), jnp.float32),
        pltpu.VMEM((_CHUNK, _DIM), jnp.float32),
        pltpu.SemaphoreType.DMA,
    ],
)
def _gather_dropout(w_hbm, ids_hbm, scale_hbm, out_hbm,
                    idx_v, rows_v, scale_v, sem):
    wid = lax.axis_index("s") * _NC + lax.axis_index("c")
    base = wid * _BPW
    pltpu.sync_copy(ids_hbm.at[wid], idx_v)

    def chunk(g, carry):
        off = base + g * _CHUNK
        pltpu.async_copy(w_hbm.at[idx_v.at[g]], rows_v, sem).wait()
        pltpu.sync_copy(scale_hbm.at[pl.ds(off, _CHUNK)], scale_v)

        def row(r, c2):
            for k in range(_DIM // 16):
                sl = pl.ds(k * 16, 16)
                rows_v[r, sl] = rows_v[r, sl] * scale_v[r, sl]
            return c2

        lax.fori_loop(0, _CHUNK, row, 0)
        pltpu.sync_copy(rows_v, out_hbm.at[pl.ds(off, _CHUNK)])
        return carry

    lax.fori_loop(0, _NCHUNK, chunk, 0)


def kernel(inputs, w):
    ids = jnp.reshape(inputs, (_NW, _NCHUNK, _CHUNK)).astype(jnp.int32)
    scale = jnp.asarray(_SCALE)
    out = _gather_dropout(w, ids, scale)
    return jnp.reshape(out, (_BATCH, _HIST, _DIM))


# speedup vs baseline: 2.8787x; 1.0390x over previous
import functools

import numpy as np
import jax
import jax.numpy as jnp
from jax import lax
from jax.experimental import pallas as pl
from jax.experimental.pallas import tpu as pltpu
from jax.experimental.pallas import tpu_sc as plsc

_VOCAB = 100000
_NUM_DEPEND = 1000
_DIM = 64
_NUM_ROWS = 1 + _VOCAB + (_NUM_DEPEND + 1)
_RATE = 0.1
_KEEP = 1.0 - _RATE
_BATCH = 16384
_HIST = 50
_TOTAL = _BATCH * _HIST

_NC = 2
_NS = 16
_NW = _NC * _NS
_BPW = _TOTAL // _NW
_CHUNK = 128
_NCHUNK = _BPW // _CHUNK


def _np_threefry2x32(k0, k1, x0, x1):
    def rotl(x, d):
        return ((x << np.uint32(d)) | (x >> np.uint32(32 - d))).astype(np.uint32)
    rot = [(13, 15, 26, 6), (17, 29, 16, 24)]
    ks = [np.uint32(k0), np.uint32(k1),
          np.uint32(k0) ^ np.uint32(k1) ^ np.uint32(0x1BD11BDA)]
    x = [(x0 + ks[0]).astype(np.uint32), (x1 + ks[1]).astype(np.uint32)]
    order = [(rot[0], 1, 2, 1), (rot[1], 2, 0, 2), (rot[0], 0, 1, 3),
             (rot[1], 1, 2, 4), (rot[0], 2, 0, 5)]
    for rots, a, b, c in order:
        for r in rots:
            x[0] = (x[0] + x[1]).astype(np.uint32)
            x[1] = x[0] ^ rotl(x[1], r)
        x[0] = (x[0] + ks[a]).astype(np.uint32)
        x[1] = (x[1] + ks[b] + np.uint32(c)).astype(np.uint32)
    return x[0], x[1]


def _dropout_scale() -> np.ndarray:
    n = _TOTAL * _DIM
    i = np.arange(n, dtype=np.uint64)
    hi = (i >> np.uint64(32)).astype(np.uint32)
    lo = (i & np.uint64(0xFFFFFFFF)).astype(np.uint32)
    b1, b2 = _np_threefry2x32(0, 42, hi, lo)
    bits = b1 ^ b2
    fb = (bits >> np.uint32(9)) | np.uint32(0x3F800000)
    u = fb.view(np.float32) - np.float32(1.0)
    keep = u < np.float32(_KEEP)
    return np.where(keep, np.float32(1.0 / _KEEP), np.float32(0.0)).reshape(
        _TOTAL, _DIM)


_SCALE = _dropout_scale()

_mesh = plsc.VectorSubcoreMesh(core_axis_name="c", subcore_axis_name="s")

_NBUF = 4
_NITER = _NCHUNK // _NBUF


@functools.partial(
    pl.kernel,
    out_type=jax.ShapeDtypeStruct((_TOTAL, _DIM), jnp.float32),
    mesh=_mesh,
    compiler_params=pltpu.CompilerParams(use_tc_tiling_on_sc=False),
    scratch_types=(
        [pltpu.VMEM((_NCHUNK, _CHUNK), jnp.int32)]
        + [pltpu.VMEM((_CHUNK, _DIM), jnp.float32)] * (3 * _NBUF)
        + [pltpu.SemaphoreType.DMA] * (3 * _NBUF)
    ),
)
def _gather_dropout(w_hbm, ids_hbm, scale_hbm, out_hbm, idx_v, *bufs):
    rows = bufs[0:_NBUF]
    res = bufs[_NBUF:2 * _NBUF]
    scl = bufs[2 * _NBUF:3 * _NBUF]
    gsem = bufs[3 * _NBUF:4 * _NBUF]
    ssem = bufs[4 * _NBUF:5 * _NBUF]
    osem = bufs[5 * _NBUF:6 * _NBUF]

    wid = lax.axis_index("s") * _NC + lax.axis_index("c")
    base = wid * _BPW
    pltpu.sync_copy(ids_hbm.at[wid], idx_v)

    for b in range(_NBUF):
        off = base + b * _CHUNK
        pltpu.async_copy(w_hbm.at[idx_v.at[b]], rows[b], gsem[b])
        pltpu.async_copy(scale_hbm.at[pl.ds(off, _CHUNK)], scl[b], ssem[b])

    def it_body(it, carry):
        for b in range(_NBUF):
            g = it * _NBUF + b
            off = base + g * _CHUNK
            pltpu.make_async_copy(w_hbm.at[idx_v.at[b]], rows[b],
                                  gsem[b]).wait()
            pltpu.make_async_copy(scale_hbm.at[pl.ds(off, _CHUNK)], scl[b],
                                  ssem[b]).wait()

            @pl.when(it > 0)
            def _wait_out():
                pltpu.make_async_copy(
                    res[b], out_hbm.at[pl.ds(off, _CHUNK)], osem[b]).wait()

            def row(r, c2):
                for k in range(_DIM // 16):
                    sl = pl.ds(k * 16, 16)
                    res[b][r, sl] = rows[b][r, sl] * scl[b][r, sl]
                return c2

            lax.fori_loop(0, _CHUNK, row, 0, unroll=2)

            g2 = g + _NBUF

            @pl.when(g2 < _NCHUNK)
            def _prefetch():
                off2 = base + g2 * _CHUNK
                pltpu.async_copy(w_hbm.at[idx_v.at[g2]], rows[b], gsem[b])
                pltpu.async_copy(scale_hbm.at[pl.ds(off2, _CHUNK)], scl[b],
                                 ssem[b])

            pltpu.async_copy(res[b], out_hbm.at[pl.ds(off, _CHUNK)], osem[b])
        return carry

    lax.fori_loop(0, _NITER, it_body, 0)

    for b in range(_NBUF):
        off = base + (_NCHUNK - _NBUF + b) * _CHUNK
        pltpu.make_async_copy(res[b], out_hbm.at[pl.ds(off, _CHUNK)],
                              osem[b]).wait()


def kernel(inputs, w):
    ids = jnp.reshape(inputs, (_NW, _NCHUNK, _CHUNK)).astype(jnp.int32)
    scale = jnp.asarray(_SCALE)
    out = _gather_dropout(w, ids, scale)
    return jnp.reshape(out, (_BATCH, _HIST, _DIM))
